# Initial kernel scaffold; baseline (speedup 1.0000x reference)
import jax
import jax.numpy as jnp
from jax.experimental import pallas as pl


def kernel(source, mode, context_id, se, me, W1, b1, W2, b2, Wh, bh):
    raise NotImplementedError("write your pallas kernel here")


# speedup vs baseline: 1.8860x; 1.8860x over previous
import jax
import jax.numpy as jnp
from jax import lax
from jax.experimental import pallas as pl
from jax.experimental.pallas import tpu as pltpu

_NB = 1024


def _trunk_body(xs_ref, xm_ref, w1_ref, b1_ref, w2_ref, b2_ref, wld_ref,
                h3_ref, p_ref):
    emb = xs_ref.shape[1]
    w1 = w1_ref[...]
    h = lax.dot_general(xs_ref[...], w1[:, :emb], (((1,), (1,)), ((), ())),
                        preferred_element_type=jnp.float32)
    h = h + lax.dot_general(xm_ref[...], w1[:, emb:], (((1,), (1,)), ((), ())),
                            preferred_element_type=jnp.float32)
    h = jnp.maximum(h + b1_ref[...], 0.0)
    h = lax.dot_general(h, w2_ref[...], (((1,), (1,)), ((), ())),
                        preferred_element_type=jnp.float32)
    h = jnp.maximum(h + b2_ref[...], 0.0)
    wld = wld_ref[...]
    nworlds = h3_ref.shape[0]
    for w in range(nworlds):
        h3_ref[w] = jnp.where(wld == w, h, 0.0).astype(jnp.bfloat16)
    widx = lax.broadcasted_iota(jnp.int32, (wld.shape[0], nworlds), 1)
    p_ref[...] = (wld == widx).astype(jnp.float32)


def _head_body(h3_ref, wh_ref, bh_ref, p_ref, out_ref):
    w = pl.program_id(1)
    hw = h3_ref[w]
    whb = wh_ref[0].astype(jnp.bfloat16)
    part = lax.dot_general(hw, whb, (((1,), (1,)), ((), ())),
                           preferred_element_type=jnp.float32)

    @pl.when(w == 0)
    def _():
        bias = lax.dot_general(p_ref[...], bh_ref[...],
                               (((1,), (0,)), ((), ())),
                               preferred_element_type=jnp.float32)
        out_ref[...] = bias + part

    @pl.when(w != 0)
    def _():
        out_ref[...] += part


def kernel(source, mode, context_id, se, me, W1, b1, W2, b2, Wh, bh):
    B = source.shape[0]
    EMB = se.shape[1]
    HID = W1.shape[0]
    NW, N, _ = Wh.shape

    xs = jnp.take(se, source, axis=0)
    xm = jnp.take(me, mode, axis=0)
    worlds = jnp.clip(context_id.astype(jnp.int32) - 1, 0, NW - 1)
    worlds = worlds.reshape(B, 1)

    h3, p = pl.pallas_call(
        _trunk_body,
        out_shape=(jax.ShapeDtypeStruct((NW, B, HID), jnp.bfloat16),
                   jax.ShapeDtypeStruct((B, NW), jnp.float32)),
    )(xs, xm, W1, b1.reshape(1, HID), W2, b2.reshape(1, HID), worlds)

    nb = _NB
    n_blocks = pl.cdiv(N, nb)
    out = pl.pallas_call(
        _head_body,
        grid=(n_blocks, NW),
        in_specs=[
            pl.BlockSpec((NW, B, HID), lambda n, w: (0, 0, 0)),
            pl.BlockSpec((1, nb, HID), lambda n, w: (w, n, 0)),
            pl.BlockSpec((NW, nb), lambda n, w: (0, n)),
            pl.BlockSpec((B, NW), lambda n, w: (0, 0)),
        ],
        out_specs=pl.BlockSpec((B, nb), lambda n, w: (0, n)),
        out_shape=jax.ShapeDtypeStruct((B, N), jnp.float32),
        compiler_params=pltpu.CompilerParams(
            dimension_semantics=("arbitrary", "arbitrary"),
        ),
    )(h3, Wh, bh, p)
    return out


# speedup vs baseline: 2.7201x; 1.4423x over previous
import jax
import jax.numpy as jnp
from jax import lax
from jax.experimental import pallas as pl
from jax.experimental.pallas import tpu as pltpu

_NB = 2048


def _trunk_body(xs_ref, xm_ref, w1_ref, b1_ref, w2_ref, b2_ref, wld_ref,
                hcat_ref, p_ref):
    emb = xs_ref.shape[1]
    hid = w2_ref.shape[0]
    w1 = w1_ref[...]
    h = lax.dot_general(xs_ref[...], w1[:, :emb], (((1,), (1,)), ((), ())),
                        preferred_element_type=jnp.float32)
    h = h + lax.dot_general(xm_ref[...], w1[:, emb:], (((1,), (1,)), ((), ())),
                            preferred_element_type=jnp.float32)
    h = jnp.maximum(h + b1_ref[...], 0.0)
    h = lax.dot_general(h, w2_ref[...], (((1,), (1,)), ((), ())),
                        preferred_element_type=jnp.float32)
    h = jnp.maximum(h + b2_ref[...], 0.0)
    hb = h.astype(jnp.bfloat16)
    wld = wld_ref[...]
    nworlds = p_ref.shape[1]
    for w in range(nworlds):
        hcat_ref[:, w * hid:(w + 1) * hid] = jnp.where(wld == w, hb, 0)
    widx = lax.broadcasted_iota(jnp.int32, (wld.shape[0], nworlds), 1)
    p_ref[...] = (wld == widx).astype(jnp.bfloat16)


def _head_body(hcat_ref, wh_ref, bh_ref, p_ref, out_ref):
    hid = wh_ref.shape[2]
    nworlds = wh_ref.shape[0]
    acc = lax.dot_general(p_ref[...], bh_ref[...].astype(jnp.bfloat16),
                          (((1,), (0,)), ((), ())),
                          preferred_element_type=jnp.float32)
    for w in range(nworlds):
        acc = acc + lax.dot_general(
            hcat_ref[:, w * hid:(w + 1) * hid],
            wh_ref[w].astype(jnp.bfloat16),
            (((1,), (1,)), ((), ())),
            preferred_element_type=jnp.float32)
    out_ref[...] = acc


def kernel(source, mode, context_id, se, me, W1, b1, W2, b2, Wh, bh):
    B = source.shape[0]
    HID = W1.shape[0]
    NW, N, _ = Wh.shape

    xs = jnp.take(se, source, axis=0)
    xm = jnp.take(me, mode, axis=0)
    worlds = jnp.clip(context_id.astype(jnp.int32) - 1, 0, NW - 1)
    worlds = worlds.reshape(B, 1)

    hcat, p = pl.pallas_call(
        _trunk_body,
        out_shape=(jax.ShapeDtypeStruct((B, NW * HID), jnp.bfloat16),
                   jax.ShapeDtypeStruct((B, NW), jnp.bfloat16)),
    )(xs, xm, W1, b1.reshape(1, HID), W2, b2.reshape(1, HID), worlds)

    nb = _NB
    n_blocks = pl.cdiv(N, nb)
    out = pl.pallas_call(
        _head_body,
        grid=(n_blocks,),
        in_specs=[
            pl.BlockSpec((B, NW * HID), lambda n: (0, 0)),
            pl.BlockSpec((NW, nb, HID), lambda n: (0, n, 0)),
            pl.BlockSpec((NW, nb), lambda n: (0, n)),
            pl.BlockSpec((B, NW), lambda n: (0, 0)),
        ],
        out_specs=pl.BlockSpec((B, nb), lambda n: (0, n)),
        out_shape=jax.ShapeDtypeStruct((B, N), jnp.float32),
        compiler_params=pltpu.CompilerParams(
            dimension_semantics=("arbitrary",),
        ),
    )(hcat, Wh, bh, p)
    return out


# speedup vs baseline: 3.2644x; 1.2001x over previous
import jax
import jax.numpy as jnp
from jax import lax
from jax.experimental import pallas as pl
from jax.experimental.pallas import tpu as pltpu

_NB = 2048


def _trunk_body(xs_ref, xm_ref, w1_ref, b1_ref, w2_ref, b2_ref, wld_ref,
                hcat_ref, p_ref):
    emb = xs_ref.shape[1]
    hid = w2_ref.shape[0]
    w1 = w1_ref[...]
    h = lax.dot_general(xs_ref[...], w1[:, :emb], (((1,), (1,)), ((), ())),
                        preferred_element_type=jnp.float32)
    h = h + lax.dot_general(xm_ref[...], w1[:, emb:], (((1,), (1,)), ((), ())),
                            preferred_element_type=jnp.float32)
    h = jnp.maximum(h + b1_ref[...], 0.0)
    h = lax.dot_general(h, w2_ref[...], (((1,), (1,)), ((), ())),
                        preferred_element_type=jnp.float32)
    h = jnp.maximum(h + b2_ref[...], 0.0)
    hb = h.astype(jnp.bfloat16)
    wld = wld_ref[...]
    nworlds = p_ref.shape[1]
    for w in range(nworlds):
        hcat_ref[:, w * hid:(w + 1) * hid] = jnp.where(wld == w, hb, 0)
    widx = lax.broadcasted_iota(jnp.int32, (wld.shape[0], nworlds), 1)
    p_ref[...] = (wld == widx).astype(jnp.bfloat16)


def _head_body(hcat_ref, wh_ref, bh_ref, p_ref, out_ref):
    hid = wh_ref.shape[2]
    nworlds = wh_ref.shape[0]
    b = hcat_ref.shape[0]
    acc = lax.dot_general(p_ref[...], bh_ref[...].astype(jnp.bfloat16),
                          (((1,), (0,)), ((), ())),
                          preferred_element_type=jnp.float32)
    whcat = jnp.concatenate(
        [wh_ref[w].astype(jnp.bfloat16) for w in range(nworlds)], axis=-1)
    acc = acc + lax.dot_general(hcat_ref[...], whcat, (((1,), (1,)), ((), ())),
                                preferred_element_type=jnp.float32)
    out_ref[...] = acc


def kernel(source, mode, context_id, se, me, W1, b1, W2, b2, Wh, bh):
    B = source.shape[0]
    HID = W1.shape[0]
    NW, N, _ = Wh.shape

    xs = jnp.take(se, source, axis=0)
    xm = jnp.take(me, mode, axis=0)
    worlds = jnp.clip(context_id.astype(jnp.int32) - 1, 0, NW - 1)
    worlds = worlds.reshape(B, 1)

    hcat, p = pl.pallas_call(
        _trunk_body,
        out_shape=(jax.ShapeDtypeStruct((B, NW * HID), jnp.bfloat16),
                   jax.ShapeDtypeStruct((B, NW), jnp.bfloat16)),
    )(xs, xm, W1, b1.reshape(1, HID), W2, b2.reshape(1, HID), worlds)

    nb = _NB
    n_blocks = pl.cdiv(N, nb)
    out = pl.pallas_call(
        _head_body,
        grid=(n_blocks,),
        in_specs=[
            pl.BlockSpec((B, NW * HID), lambda n: (0, 0)),
            pl.BlockSpec((NW, nb, HID), lambda n: (0, n, 0)),
            pl.BlockSpec((NW, nb), lambda n: (0, n)),
            pl.BlockSpec((B, NW), lambda n: (0, 0)),
        ],
        out_specs=pl.BlockSpec((B, nb), lambda n: (0, n)),
        out_shape=jax.ShapeDtypeStruct((B, N), jnp.float32),
        compiler_params=pltpu.CompilerParams(
            dimension_semantics=("arbitrary",),
        ),
    )(hcat, Wh, bh, p)
    return out
